# baseline (speedup 1.0000x reference)
import functools

import jax
import jax.numpy as jnp
from jax import lax
from jax.experimental import pallas as pl
from jax.experimental.pallas import tpu as pltpu
from jax.experimental.pallas import tpu_sc as plsc

N_TOK = 16384
D_IN = 2048
G_GRP = 8
E_PER_G = 8
E_TOT = G_GRP * E_PER_G
F_PAD = 80
BLK = 1024

NC = 2
NS = 16
NW = NC * NS
TOK_W = N_TOK // NW
LANES = 16
CHUNKS = TOK_W // LANES


def _gemm_block(x_ref, w_ref, z_ref):
    z_ref[...] = jnp.dot(x_ref[...], w_ref[...],
                         preferred_element_type=jnp.float32,
                         precision=jax.lax.Precision.DEFAULT)


def _tc_logits(x, wct):
    return pl.pallas_call(
        _gemm_block,
        grid=(N_TOK // BLK,),
        in_specs=[
            pl.BlockSpec((BLK, D_IN), lambda i: (i, 0)),
            pl.BlockSpec((D_IN, F_PAD), lambda i: (0, 0)),
        ],
        out_specs=pl.BlockSpec((BLK, F_PAD), lambda i: (i, 0)),
        out_shape=jax.ShapeDtypeStruct((N_TOK, F_PAD), jnp.float32),
    )(x, wct)


_SC_MESH = plsc.VectorSubcoreMesh(core_axis_name="c", subcore_axis_name="s")


@functools.partial(
    pl.kernel,
    mesh=_SC_MESH,
    compiler_params=pltpu.CompilerParams(needs_layout_passes=False),
    out_type=[
        jax.ShapeDtypeStruct((N_TOK * E_TOT,), jnp.int32),
        jax.ShapeDtypeStruct((N_TOK * E_TOT,), jnp.float32),
    ],
    scratch_types=[
        pltpu.VMEM((TOK_W * F_PAD,), jnp.float32),
        pltpu.VMEM((TOK_W * E_TOT,), jnp.int32),
        pltpu.VMEM((TOK_W * E_TOT,), jnp.float32),
    ],
)
def _sc_router(z_hbm, mask_hbm, nw_hbm, z_v, mask_v, nw_v):
    wid = lax.axis_index("s") * NC + lax.axis_index("c")
    tok0 = wid * TOK_W
    pltpu.sync_copy(z_hbm.at[pl.ds(tok0 * F_PAD, TOK_W * F_PAD)], z_v)
    lane = lax.iota(jnp.int32, LANES)

    @plsc.parallel_loop(0, CHUNKS, 1, unroll=2)
    def chunk(t):
        rows = t * LANES + lane
        zbase = rows * F_PAD
        obase = rows * E_TOT

        def feat(j):
            return plsc.load_gather(z_v, [zbase + j])

        ge = [jnp.exp(feat(E_TOT + g)) for g in range(G_GRP)]
        gsum = ge[0]
        for g in range(1, G_GRP):
            gsum = gsum + ge[g]
        grec = 1.0 / gsum

        wvals = []
        vvals = []
        wsum = jnp.zeros((LANES,), jnp.float32)
        for g in range(G_GRP):
            gp = ge[g] * grec
            gm = gp >= 0.125
            es = [jnp.exp(feat(g * E_PER_G + k)) for k in range(E_PER_G)]
            esum = es[0]
            for k in range(1, E_PER_G):
                esum = esum + es[k]
            erec = 1.0 / esum
            for k in range(E_PER_G):
                ep = es[k] * erec
                valid = gm & (ep >= 0.125)
                w = jnp.where(valid, gp * ep, 0.0)
                wsum = wsum + w
                wvals.append(w)
                vvals.append(valid)

        wrec = 1.0 / jnp.maximum(wsum, 1e-9)
        for j in range(E_TOT):
            plsc.store_scatter(mask_v, [obase + j],
                               vvals[j].astype(jnp.int32))
            plsc.store_scatter(nw_v, [obase + j], wvals[j] * wrec)

    pltpu.sync_copy(mask_v, mask_hbm.at[pl.ds(tok0 * E_TOT, TOK_W * E_TOT)])
    pltpu.sync_copy(nw_v, nw_hbm.at[pl.ds(tok0 * E_TOT, TOK_W * E_TOT)])


@jax.jit
def kernel(x, Wg, We):
    wct = jnp.concatenate(
        [We, Wg, jnp.zeros((F_PAD - E_TOT - G_GRP, D_IN), jnp.float32)],
        axis=0).T
    z = _tc_logits(x, wct)
    mask_i32, nw = _sc_router(z.reshape(N_TOK * F_PAD))
    mask = mask_i32.reshape(N_TOK, E_TOT).astype(jnp.bool_)
    return mask, nw.reshape(N_TOK, E_TOT)


# speedup vs baseline: 1.0385x; 1.0385x over previous
import functools

import jax
import jax.numpy as jnp
from jax import lax
from jax.experimental import pallas as pl
from jax.experimental.pallas import tpu as pltpu
from jax.experimental.pallas import tpu_sc as plsc

N_TOK = 16384
D_IN = 2048
G_GRP = 8
E_PER_G = 8
E_TOT = G_GRP * E_PER_G
F_PAD = 80
BLK = 1024

N_STAGE = 2
H_TOK = N_TOK // N_STAGE

NC = 2
NS = 16
NW = NC * NS
TOK_W = H_TOK // NW
LANES = 16
CHUNKS = TOK_W // LANES


def _gemm_block(x_ref, w_ref, z_ref):
    z_ref[...] = jnp.dot(x_ref[...], w_ref[...],
                         preferred_element_type=jnp.float32,
                         precision=jax.lax.Precision.DEFAULT)


def _tc_logits(x, wct, h):
    nblk = H_TOK // BLK
    return pl.pallas_call(
        _gemm_block,
        grid=(nblk,),
        in_specs=[
            pl.BlockSpec((BLK, D_IN), lambda i, _h=h, _n=nblk: (i + _h * _n, 0)),
            pl.BlockSpec((D_IN, F_PAD), lambda i: (0, 0)),
        ],
        out_specs=pl.BlockSpec((BLK, F_PAD), lambda i: (i, 0)),
        out_shape=jax.ShapeDtypeStruct((H_TOK, F_PAD), jnp.float32),
    )(x, wct)


_SC_MESH = plsc.VectorSubcoreMesh(core_axis_name="c", subcore_axis_name="s")


@functools.partial(
    pl.kernel,
    mesh=_SC_MESH,
    compiler_params=pltpu.CompilerParams(needs_layout_passes=False),
    out_type=[
        jax.ShapeDtypeStruct((H_TOK * E_TOT,), jnp.int32),
        jax.ShapeDtypeStruct((H_TOK * E_TOT,), jnp.float32),
    ],
    scratch_types=[
        pltpu.VMEM((TOK_W * F_PAD,), jnp.float32),
        pltpu.VMEM((TOK_W * E_TOT,), jnp.int32),
        pltpu.VMEM((TOK_W * E_TOT,), jnp.float32),
    ],
)
def _sc_router(z_hbm, mask_hbm, nw_hbm, z_v, mask_v, nw_v):
    wid = lax.axis_index("s") * NC + lax.axis_index("c")
    tok0 = wid * TOK_W
    pltpu.sync_copy(z_hbm.at[pl.ds(tok0 * F_PAD, TOK_W * F_PAD)], z_v)
    lane = lax.iota(jnp.int32, LANES)

    @plsc.parallel_loop(0, CHUNKS, 1, unroll=2)
    def chunk(t):
        rows = t * LANES + lane
        zbase = rows * F_PAD
        obase = rows * E_TOT

        def feat(j):
            return plsc.load_gather(z_v, [zbase + j])

        ge = [jnp.exp(feat(E_TOT + g)) for g in range(G_GRP)]
        gsum = ge[0]
        for g in range(1, G_GRP):
            gsum = gsum + ge[g]
        grec = 1.0 / gsum

        wvals = []
        vvals = []
        wsum = jnp.zeros((LANES,), jnp.float32)
        for g in range(G_GRP):
            gp = ge[g] * grec
            gm = gp >= 0.125
            es = [jnp.exp(feat(g * E_PER_G + k)) for k in range(E_PER_G)]
            esum = es[0]
            for k in range(1, E_PER_G):
                esum = esum + es[k]
            erec = 1.0 / esum
            for k in range(E_PER_G):
                ep = es[k] * erec
                valid = gm & (ep >= 0.125)
                w = jnp.where(valid, gp * ep, 0.0)
                wsum = wsum + w
                wvals.append(w)
                vvals.append(valid)

        wrec = 1.0 / jnp.maximum(wsum, 1e-9)
        for j in range(E_TOT):
            plsc.store_scatter(mask_v, [obase + j],
                               vvals[j].astype(jnp.int32))
            plsc.store_scatter(nw_v, [obase + j], wvals[j] * wrec)

    pltpu.sync_copy(mask_v, mask_hbm.at[pl.ds(tok0 * E_TOT, TOK_W * E_TOT)])
    pltpu.sync_copy(nw_v, nw_hbm.at[pl.ds(tok0 * E_TOT, TOK_W * E_TOT)])


@jax.jit
def kernel(x, Wg, We):
    wct = jnp.concatenate(
        [We, Wg, jnp.zeros((F_PAD - E_TOT - G_GRP, D_IN), jnp.float32)],
        axis=0).T
    masks = []
    nws = []
    for h in range(N_STAGE):
        z = _tc_logits(x, wct, h)
        m, w = _sc_router(z.reshape(H_TOK * F_PAD))
        masks.append(m.reshape(H_TOK, E_TOT))
        nws.append(w.reshape(H_TOK, E_TOT))
    mask = jnp.concatenate(masks, axis=0).astype(jnp.bool_)
    nw = jnp.concatenate(nws, axis=0)
    return mask, nw


# speedup vs baseline: 1.0411x; 1.0026x over previous
import functools

import jax
import jax.numpy as jnp
from jax import lax
from jax.experimental import pallas as pl
from jax.experimental.pallas import tpu as pltpu
from jax.experimental.pallas import tpu_sc as plsc

N_TOK = 16384
D_IN = 2048
G_GRP = 8
E_PER_G = 8
E_TOT = G_GRP * E_PER_G
F_PAD = 80
BLK = 1024

N_STAGE = 2
H_TOK = N_TOK // N_STAGE

NC = 2
NS = 16
NW = NC * NS
TOK_W = H_TOK // NW
LANES = 16
CHUNKS = TOK_W // LANES


def _gemm_block(x_ref, w_ref, z_ref):
    z_ref[...] = jnp.dot(x_ref[...], w_ref[...],
                         preferred_element_type=jnp.float32,
                         precision=jax.lax.Precision.DEFAULT)


def _tc_logits(x, wct, h):
    nblk = H_TOK // BLK
    return pl.pallas_call(
        _gemm_block,
        grid=(nblk,),
        in_specs=[
            pl.BlockSpec((BLK, D_IN), lambda i, _h=h, _n=nblk: (i + _h * _n, 0)),
            pl.BlockSpec((D_IN, F_PAD), lambda i: (0, 0)),
        ],
        out_specs=pl.BlockSpec((BLK, F_PAD), lambda i: (i, 0)),
        out_shape=jax.ShapeDtypeStruct((H_TOK, F_PAD), jnp.float32),
    )(x, wct)


_SC_MESH = plsc.VectorSubcoreMesh(core_axis_name="c", subcore_axis_name="s")


@functools.partial(
    pl.kernel,
    mesh=_SC_MESH,
    compiler_params=pltpu.CompilerParams(needs_layout_passes=False),
    out_type=[
        jax.ShapeDtypeStruct((H_TOK * E_TOT,), jnp.int32),
        jax.ShapeDtypeStruct((H_TOK * E_TOT,), jnp.float32),
    ],
    scratch_types=[
        pltpu.VMEM((TOK_W * F_PAD,), jnp.float32),
        pltpu.VMEM((TOK_W * E_TOT,), jnp.int32),
        pltpu.VMEM((TOK_W * E_TOT,), jnp.float32),
    ],
)
def _sc_router(z_hbm, mask_hbm, nw_hbm, z_v, mask_v, nw_v):
    wid = lax.axis_index("s") * NC + lax.axis_index("c")
    tok0 = wid * TOK_W
    pltpu.sync_copy(z_hbm.at[pl.ds(tok0 * F_PAD, TOK_W * F_PAD)], z_v)
    lane = lax.iota(jnp.int32, LANES)

    @plsc.parallel_loop(0, CHUNKS, 1, unroll=2)
    def chunk(t):
        rows = t * LANES + lane
        zbase = rows * F_PAD
        obase = rows * E_TOT

        def feat(j):
            return plsc.load_gather(z_v, [zbase + j])

        ge = [jnp.exp(feat(E_TOT + g)) for g in range(G_GRP)]
        gsum = ge[0]
        for g in range(1, G_GRP):
            gsum = gsum + ge[g]
        grec = 1.0 / gsum

        wvals = []
        vvals = []
        wsum = jnp.zeros((LANES,), jnp.float32)
        for g in range(G_GRP):
            gp = ge[g] * grec
            gm = gp >= 0.125
            es = [jnp.exp(feat(g * E_PER_G + k)) for k in range(E_PER_G)]
            esum = es[0]
            for k in range(1, E_PER_G):
                esum = esum + es[k]
            erec = 1.0 / esum
            for k in range(E_PER_G):
                ep = es[k] * erec
                valid = gm & (ep >= 0.125)
                w = jnp.where(valid, gp * ep, 0.0)
                wsum = wsum + w
                wvals.append(w)
                vvals.append(valid)

        wrec = 1.0 / jnp.maximum(wsum, 1e-9)
        for j in range(E_TOT):
            plsc.store_scatter(mask_v, [obase + j],
                               vvals[j].astype(jnp.int32))
            plsc.store_scatter(nw_v, [obase + j], wvals[j] * wrec)

    pltpu.sync_copy(mask_v, mask_hbm.at[pl.ds(tok0 * E_TOT, TOK_W * E_TOT)])
    pltpu.sync_copy(nw_v, nw_hbm.at[pl.ds(tok0 * E_TOT, TOK_W * E_TOT)])


@jax.jit
def kernel(x, Wg, We):
    wct = jnp.concatenate(
        [We, Wg, jnp.zeros((F_PAD - E_TOT - G_GRP, D_IN), jnp.float32)],
        axis=0).T
    zs = [_tc_logits(x, wct, h) for h in range(N_STAGE)]
    masks = []
    nws = []
    for z in zs:
        m, w = _sc_router(z.reshape(H_TOK * F_PAD))
        masks.append(m.reshape(H_TOK, E_TOT))
        nws.append(w.reshape(H_TOK, E_TOT))
    mask = jnp.concatenate(masks, axis=0).astype(jnp.bool_)
    nw = jnp.concatenate(nws, axis=0)
    return mask, nw


# speedup vs baseline: 1.1951x; 1.1479x over previous
import functools

import jax
import jax.numpy as jnp
from jax import lax
from jax.experimental import pallas as pl
from jax.experimental.pallas import tpu as pltpu
from jax.experimental.pallas import tpu_sc as plsc

N_TOK = 16384
D_IN = 2048
G_GRP = 8
E_PER_G = 8
E_TOT = G_GRP * E_PER_G
F_PAD = 80
BLK = 1024

NC = 2
NS = 16
NW = NC * NS
TOK_W = N_TOK // NW
LANES = 16
CHUNKS = TOK_W // LANES


def _gemm_exp_block(x_ref, w_ref, e_ref):
    z = jnp.dot(x_ref[...], w_ref[...],
                preferred_element_type=jnp.float32,
                precision=jax.lax.Precision.DEFAULT)
    e_ref[...] = jnp.exp(z)


def _tc_exp_logits(x, wct):
    return pl.pallas_call(
        _gemm_exp_block,
        grid=(N_TOK // BLK,),
        in_specs=[
            pl.BlockSpec((BLK, D_IN), lambda i: (i, 0)),
            pl.BlockSpec((D_IN, F_PAD), lambda i: (0, 0)),
        ],
        out_specs=pl.BlockSpec((BLK, F_PAD), lambda i: (i, 0)),
        out_shape=jax.ShapeDtypeStruct((N_TOK, F_PAD), jnp.float32),
    )(x, wct)


_SC_MESH = plsc.VectorSubcoreMesh(core_axis_name="c", subcore_axis_name="s")


@functools.partial(
    pl.kernel,
    mesh=_SC_MESH,
    compiler_params=pltpu.CompilerParams(needs_layout_passes=False),
    out_type=jax.ShapeDtypeStruct((N_TOK * E_TOT,), jnp.float32),
    scratch_types=[
        pltpu.VMEM((TOK_W * F_PAD,), jnp.float32),
        pltpu.VMEM((TOK_W * E_TOT,), jnp.float32),
    ],
)
def _sc_router(e_hbm, nw_hbm, e_v, nw_v):
    wid = lax.axis_index("s") * NC + lax.axis_index("c")
    tok0 = wid * TOK_W
    pltpu.sync_copy(e_hbm.at[pl.ds(tok0 * F_PAD, TOK_W * F_PAD)], e_v)
    lane = lax.iota(jnp.int32, LANES)

    @plsc.parallel_loop(0, CHUNKS, 1, unroll=2)
    def chunk(t):
        rows = t * LANES + lane
        ebase = rows * F_PAD
        obase = rows * E_TOT

        def feat(j):
            return plsc.load_gather(e_v, [ebase + j])

        ge = [feat(E_TOT + g) for g in range(G_GRP)]
        gsum = ge[0]
        for g in range(1, G_GRP):
            gsum = gsum + ge[g]
        grec = 1.0 / gsum

        wvals = []
        wsum = jnp.zeros((LANES,), jnp.float32)
        for g in range(G_GRP):
            gp = ge[g] * grec
            gm = gp >= 0.125
            es = [feat(g * E_PER_G + k) for k in range(E_PER_G)]
            esum = es[0]
            for k in range(1, E_PER_G):
                esum = esum + es[k]
            erec = 1.0 / esum
            for k in range(E_PER_G):
                ep = es[k] * erec
                valid = gm & (ep >= 0.125)
                w = jnp.where(valid, gp * ep, 0.0)
                wsum = wsum + w
                wvals.append(w)

        wrec = 1.0 / jnp.maximum(wsum, 1e-9)
        for j in range(E_TOT):
            plsc.store_scatter(nw_v, [obase + j], wvals[j] * wrec)

    pltpu.sync_copy(nw_v, nw_hbm.at[pl.ds(tok0 * E_TOT, TOK_W * E_TOT)])


@jax.jit
def kernel(x, Wg, We):
    wct = jnp.concatenate(
        [We, Wg, jnp.zeros((F_PAD - E_TOT - G_GRP, D_IN), jnp.float32)],
        axis=0).T
    e = _tc_exp_logits(x, wct)
    nw = _sc_router(e.reshape(N_TOK * F_PAD)).reshape(N_TOK, E_TOT)
    return nw > 0.0, nw
